# baseline (speedup 1.0000x reference)
import jax
import jax.numpy as jnp
from jax import lax
from jax.experimental import pallas as pl
from jax.experimental.pallas import tpu as pltpu
from jax.experimental.pallas import tpu_sc as plsc

_NB = 64
_DM = 128
_B = 4096
_S = 200
_TOT = _B * _S
_NC = 2
_NS = 16
_NW = _NC * _NS
_PER_W = _TOT // _NW
_CHUNK = 128
_NCHUNK = _PER_W // _CHUNK
_NBUF = 5
_LAG = 2
_T = _NCHUNK // _NBUF
assert _NCHUNK % _NBUF == 0 and _LAG < _NBUF


def _emb_body(idx_hbm, table_hbm, rep_hbm, out_hbm, idx_v, table_v, rows, sem_g, sem_s):
    c = lax.axis_index("c")
    s = lax.axis_index("s")
    wid = s * _NC + c
    base = wid * _PER_W
    pltpu.sync_copy(idx_hbm.at[pl.ds(base, _PER_W)], idx_v)

    @pl.when(s == 0)
    def _():
        pltpu.sync_copy(table_hbm, table_v)

    plsc.subcore_barrier()

    def gather(g, b):
        if b == _NBUF - 1:
            src = rep_hbm.at[wid]
        else:
            src = table_v
        pltpu.async_copy(
            src.at[idx_v.at[pl.ds(g * _CHUNK, _CHUNK)]], rows[b], sem_g[b]
        )

    def store(g, b):
        pltpu.async_copy(
            rows[b], out_hbm.at[pl.ds(base + g * _CHUNK, _CHUNK)], sem_s[b]
        )

    def wait_gather(b):
        pltpu.make_async_copy(
            table_v.at[idx_v.at[pl.ds(0, _CHUNK)]], rows[b], sem_g[b]
        ).wait()

    def wait_store(b):
        pltpu.make_async_copy(
            rows[b], out_hbm.at[pl.ds(base, _CHUNK)], sem_s[b]
        ).wait()

    def outer(t, carry):
        g0 = t * _NBUF
        for b in range(_NBUF):
            g = g0 + b

            @pl.when(t > 0)
            def _():
                wait_store(b)

            gather(g, b)

            h = g - _LAG
            bh = (b - _LAG) % _NBUF

            @pl.when(h >= 0)
            def _():
                wait_gather(bh)
                store(h, bh)

        return carry

    lax.fori_loop(0, _T, outer, 0)

    for h in range(_NCHUNK - _LAG, _NCHUNK):
        bh = h % _NBUF
        wait_gather(bh)
        store(h, bh)
    for h in range(_NCHUNK - _NBUF, _NCHUNK):
        wait_store(h % _NBUF)


def kernel(band_idx, table):
    idx = band_idx.reshape(_TOT).astype(jnp.int32)
    mesh = plsc.VectorSubcoreMesh(core_axis_name="c", subcore_axis_name="s")
    out = pl.kernel(
        _emb_body,
        out_type=jax.ShapeDtypeStruct((_TOT, _DM), jnp.float32),
        mesh=mesh,
        scratch_types=[
            pltpu.VMEM((_PER_W,), jnp.int32),
            pltpu.VMEM_SHARED((_NB, _DM), jnp.float32),
            [pltpu.VMEM((_CHUNK, _DM), jnp.float32) for _ in range(_NBUF)],
            [pltpu.SemaphoreType.DMA for _ in range(_NBUF)],
            [pltpu.SemaphoreType.DMA for _ in range(_NBUF)],
        ],
    )(idx, table, jnp.broadcast_to(table, (_NW, _NB, _DM)))
    return out.reshape(_B, _S, _DM)


# speedup vs baseline: 1.2283x; 1.2283x over previous
import jax
import jax.numpy as jnp
from jax import lax
from jax.experimental import pallas as pl
from jax.experimental.pallas import tpu as pltpu
from jax.experimental.pallas import tpu_sc as plsc

_NB = 64
_DM = 128
_B = 4096
_S = 200
_TOT = _B * _S
_NC = 2
_NS = 16
_NW = _NC * _NS
_PER_W = _TOT // _NW
_CHUNK = 128
_NCHUNK = _PER_W // _CHUNK
_NBUF = 5
_LAG = 2
_T = _NCHUNK // _NBUF
assert _NCHUNK % _NBUF == 0 and _LAG < _NBUF


def _emb_body(idx_hbm, table_hbm, out_hbm, idx_v, table_v, rows, sem_g, sem_s):
    c = lax.axis_index("c")
    s = lax.axis_index("s")
    wid = s * _NC + c
    base = wid * _PER_W
    pltpu.sync_copy(idx_hbm.at[pl.ds(base, _PER_W)], idx_v)

    @pl.when(s == 0)
    def _():
        pltpu.sync_copy(table_hbm, table_v)

    plsc.subcore_barrier()

    def gather(g, b):
        pltpu.async_copy(
            table_v.at[idx_v.at[pl.ds(g * _CHUNK, _CHUNK)]], rows[b], sem_g[b]
        )

    def store(g, b):
        pltpu.async_copy(
            rows[b], out_hbm.at[pl.ds(base + g * _CHUNK, _CHUNK)], sem_s[b]
        )

    def wait_gather(b):
        pltpu.make_async_copy(
            table_v.at[idx_v.at[pl.ds(0, _CHUNK)]], rows[b], sem_g[b]
        ).wait()

    def wait_store(b):
        pltpu.make_async_copy(
            rows[b], out_hbm.at[pl.ds(base, _CHUNK)], sem_s[b]
        ).wait()

    def outer(t, carry):
        g0 = t * _NBUF
        for b in range(_NBUF):
            g = g0 + b

            @pl.when(t > 0)
            def _():
                wait_store(b)

            gather(g, b)

            h = g - _LAG
            bh = (b - _LAG) % _NBUF

            @pl.when(h >= 0)
            def _():
                wait_gather(bh)
                store(h, bh)

        return carry

    lax.fori_loop(0, _T, outer, 0)

    for h in range(_NCHUNK - _LAG, _NCHUNK):
        bh = h % _NBUF
        wait_gather(bh)
        store(h, bh)
    for h in range(_NCHUNK - _NBUF, _NCHUNK):
        wait_store(h % _NBUF)


def kernel(band_idx, table):
    idx = band_idx.reshape(_TOT).astype(jnp.int32)
    mesh = plsc.VectorSubcoreMesh(core_axis_name="c", subcore_axis_name="s")
    out = pl.kernel(
        _emb_body,
        out_type=jax.ShapeDtypeStruct((_TOT, _DM), jnp.float32),
        mesh=mesh,
        scratch_types=[
            pltpu.VMEM((_PER_W,), jnp.int32),
            pltpu.VMEM_SHARED((_NB, _DM), jnp.float32),
            [pltpu.VMEM((_CHUNK, _DM), jnp.float32) for _ in range(_NBUF)],
            [pltpu.SemaphoreType.DMA for _ in range(_NBUF)],
            [pltpu.SemaphoreType.DMA for _ in range(_NBUF)],
        ],
    )(idx, table)
    return out.reshape(_B, _S, _DM)
